# baseline (speedup 1.0000x reference)
import jax
import jax.numpy as jnp
from jax import lax
from jax.experimental import pallas as pl
from jax.experimental.pallas import tpu as pltpu
from jax.experimental.pallas import tpu_sc as plsc

_S = 50
_B = 4096
_D = 300
_SP = 64
_NC = 2
_NS = 16
_NW = _NC * _NS
_COLS = _B // _NW
_VP = 196 * 512

_CHUNKS = [tuple(16 * j for j in range(8)),
           tuple(16 * j for j in range(8)),
           (0, 16, 28)]
_WIDTHS = (128, 128, 44)


def _make_sc_pool_body(k):
    chunks = _CHUNKS[k]

    def body(tk_hbm, xt_hbm, xg_hbm, out_hbm,
             xt_blk, xg_blk, rows_a, rows_b, out_blk, sem_a, sem_b):
        wid = lax.axis_index("s") * _NC + lax.axis_index("c")
        base = wid * _COLS
        zeros16 = jnp.zeros((16,), jnp.float32)
        ones16 = jnp.full((16,), 1.0, jnp.float32)

        pltpu.sync_copy(xt_hbm.at[pl.ds(base, _COLS)], xt_blk)
        pltpu.sync_copy(xg_hbm.at[pl.ds(base, _COLS)], xg_blk)
        if k == 2:
            zoffs = [s + d for s in (0, 128, 256)
                     for d in (32, 48, 64, 80, 96, 112)]
            for r in range(16):
                for z in zoffs:
                    out_blk[r, pl.ds(z, 16)] = zeros16

        def fetch(c, rows, sem):
            pltpu.make_async_copy(tk_hbm.at[xg_blk.at[c]], rows, sem).start()

        def compute(c, rows, sem):
            pltpu.make_async_copy(tk_hbm.at[xg_blk.at[c]], rows, sem).wait()
            r = lax.rem(c, 16)
            cnt = jnp.zeros((16,), jnp.float32)
            for q in range(4):
                cnt = cnt + jnp.where(xt_blk[c, pl.ds(16 * q, 16)] != 1,
                                      ones16, zeros16)
            inv = ones16 / jnp.full((16,), jnp.sum(cnt))
            for off in chunks:
                def lbody(i, carry, off=off):
                    acc = list(carry)
                    for u in range(10):
                        v = rows[i * 10 + u, pl.ds(off, 16)]
                        q = u % 4
                        acc[q] = acc[q] + v
                        acc[4 + q] = jnp.maximum(acc[4 + q], v)
                        acc[8 + q] = jnp.minimum(acc[8 + q], v)
                    return tuple(acc)

                ninf16 = jnp.full((16,), -jnp.inf, jnp.float32)
                pinf16 = jnp.full((16,), jnp.inf, jnp.float32)
                init = (zeros16,) * 4 + (ninf16,) * 4 + (pinf16,) * 4
                acc = lax.fori_loop(0, _S // 10, lbody, init)
                a_s = (acc[0] + acc[1]) + (acc[2] + acc[3])
                a_mx = jnp.maximum(jnp.maximum(acc[4], acc[5]),
                                   jnp.maximum(acc[6], acc[7]))
                a_mn = jnp.minimum(jnp.minimum(acc[8], acc[9]),
                                   jnp.minimum(acc[10], acc[11]))
                out_blk[r, pl.ds(off, 16)] = a_mx
                out_blk[r, pl.ds(128 + off, 16)] = a_s * inv
                out_blk[r, pl.ds(256 + off, 16)] = a_mn

        fetch(0, rows_a, sem_a)

        def loop_body(it, carry):
            c0 = 2 * it
            fetch(c0 + 1, rows_b, sem_b)
            compute(c0, rows_a, sem_a)

            @pl.when(it < _COLS // 2 - 1)
            def _():
                fetch(c0 + 2, rows_a, sem_a)

            compute(c0 + 1, rows_b, sem_b)

            @pl.when(lax.rem(it, 8) == 7)
            def _():
                grp = lax.div(it, 8)
                pltpu.sync_copy(out_blk,
                                out_hbm.at[pl.ds(base + grp * 16, 16)])

            return carry

        lax.fori_loop(0, _COLS // 2, loop_body, 0)

    return body


def _sc_pool_plane(tk, xtp, xgp, k):
    mesh = plsc.VectorSubcoreMesh(core_axis_name="c", subcore_axis_name="s")
    f = pl.kernel(
        _make_sc_pool_body(k),
        out_type=jax.ShapeDtypeStruct((_B, 384), jnp.float32),
        mesh=mesh,
        compiler_params=pltpu.CompilerParams(use_tc_tiling_on_sc=False,
                                             needs_layout_passes=False),
        scratch_types=[
            pltpu.VMEM((_COLS, _SP), jnp.int32),
            pltpu.VMEM((_COLS, _S), jnp.int32),
            pltpu.VMEM((_S, 128), jnp.float32),
            pltpu.VMEM((_S, 128), jnp.float32),
            pltpu.VMEM((16, 384), jnp.float32),
            pltpu.SemaphoreType.DMA,
            pltpu.SemaphoreType.DMA,
        ],
        name=f"sc_pool_p{k}",
    )
    return f(tk, xtp, xgp)


def _tc_transpose_plane_body(i_ref, o_ref):
    o_ref[...] = i_ref[...].T


def _tc_transpose_plane(tt, k):
    return pl.pallas_call(
        _tc_transpose_plane_body,
        grid=(196,),
        in_specs=[pl.BlockSpec((128, 512), lambda i, k=k: (k, i))],
        out_specs=pl.BlockSpec((512, 128), lambda i: (i, 0)),
        out_shape=jax.ShapeDtypeStruct((_VP, 128), jnp.float32),
        name=f"transpose_p{k}",
    )(tt)


def _tc_head_body(p0_ref, p1_ref, p2_ref, ag_ref,
                  w0_ref, w1_ref, w2_ref, wd_ref, b_ref, o_ref):
    acc = jnp.dot(p0_ref[...], w0_ref[...], preferred_element_type=jnp.float32)
    acc = acc + jnp.dot(p1_ref[...], w1_ref[...],
                        preferred_element_type=jnp.float32)
    acc = acc + jnp.dot(p2_ref[...], w2_ref[...],
                        preferred_element_type=jnp.float32)
    acc = acc + jnp.dot(ag_ref[...], wd_ref[...],
                        preferred_element_type=jnp.float32)
    acc = acc + b_ref[...]
    cols = lax.broadcasted_iota(jnp.int32, acc.shape, 1)
    acc = jnp.where(cols < 10, acc, -jnp.inf)
    m = jnp.max(acc, axis=1, keepdims=True)
    lse = jnp.log(jnp.sum(jnp.exp(acc - m), axis=1, keepdims=True)) + m
    o_ref[...] = acc - lse


def _tc_head(p0, p1, p2, ag, w0, w1, w2, wd, bp):
    pspec = pl.BlockSpec((_B // 16, 384), lambda i: (i, 0))
    wspec = pl.BlockSpec((384, 128), lambda i: (0, 0))
    return pl.pallas_call(
        _tc_head_body,
        grid=(16,),
        in_specs=[
            pspec, pspec, pspec,
            pl.BlockSpec((_B // 16, 128), lambda i: (i, 0)),
            wspec, wspec, wspec,
            pl.BlockSpec((128, 128), lambda i: (0, 0)),
            pl.BlockSpec((1, 128), lambda i: (0, 0)),
        ],
        out_specs=pl.BlockSpec((_B // 16, 128), lambda i: (i, 0)),
        out_shape=jax.ShapeDtypeStruct((_B, 128), jnp.float32),
    )(p0, p1, p2, ag, w0, w1, w2, wd, bp)


def kernel(x, age, gender, table, W, b):
    xt = x.T
    xtp = jnp.full((_B, _SP), 1, jnp.int32).at[:, :_S].set(xt)
    tt = jnp.swapaxes(table, 0, 1)
    pooled = []
    ws = []
    for k in range(3):
        tk = _tc_transpose_plane(tt, k)
        pooled.append(_sc_pool_plane(tk, xtp, xt, k))
        w = _WIDTHS[k]
        lo = 128 * k
        ws.append(jnp.zeros((384, 128), jnp.float32)
                  .at[0:w, :10].set(W[:, lo:lo + w].T)
                  .at[128:128 + w, :10].set(W[:, 300 + lo:300 + lo + w].T)
                  .at[256:256 + w, :10].set(W[:, 600 + lo:600 + lo + w].T))
    ag = (jnp.zeros((_B, 128), jnp.float32)
          .at[:, :11].set(age).at[:, 11:13].set(gender))
    wd = jnp.zeros((128, 128), jnp.float32).at[:13, :10].set(W[:, 900:].T)
    bp = jnp.zeros((1, 128), jnp.float32).at[0, :10].set(b)
    out = _tc_head(pooled[0], pooled[1], pooled[2], ag, ws[0], ws[1], ws[2],
                   wd, bp)
    return out[:, :10]


# speedup vs baseline: 1.1543x; 1.1543x over previous
import jax
import jax.numpy as jnp
from jax import lax
from jax.experimental import pallas as pl
from jax.experimental.pallas import tpu as pltpu
from jax.experimental.pallas import tpu_sc as plsc

_S = 50
_B = 4096
_D = 300
_SP = 64
_NC = 2
_NS = 16
_NW = _NC * _NS
_COLS = _B // _NW
_PD = 1152
_VP = 196 * 512


def _sc_pool_body(t0_hbm, t1_hbm, t2_hbm, xt_hbm, xg_hbm, out_hbm,
                  xt_blk, xg_blk, rows_a0, rows_a1, rows_a2,
                  rows_b0, rows_b1, rows_b2, out_blk,
                  sem_a, sem_b):
    wid = lax.axis_index("s") * _NC + lax.axis_index("c")
    base = wid * _COLS
    zeros16 = jnp.zeros((16,), jnp.float32)
    ones16 = jnp.full((16,), 1.0, jnp.float32)

    pltpu.sync_copy(xt_hbm.at[pl.ds(base, _COLS)], xt_blk)
    pltpu.sync_copy(xg_hbm.at[pl.ds(base, _COLS)], xg_blk)
    zoffs = [s + d for s in (0, 384, 768)
             for d in (300, 316, 332, 348, 364, 368)]
    for r in range(16):
        for z in zoffs:
            out_blk[r, pl.ds(z, 16)] = zeros16

    def fetch(c, rows3, sem):
        idx = xg_blk.at[c]
        pltpu.make_async_copy(t0_hbm.at[idx], rows3[0], sem).start()
        pltpu.make_async_copy(t1_hbm.at[idx], rows3[1], sem).start()
        pltpu.make_async_copy(t2_hbm.at[idx], rows3[2], sem).start()

    def compute(c, rows3, sem):
        idx = xg_blk.at[c]
        pltpu.make_async_copy(t0_hbm.at[idx], rows3[0], sem).wait()
        pltpu.make_async_copy(t1_hbm.at[idx], rows3[1], sem).wait()
        pltpu.make_async_copy(t2_hbm.at[idx], rows3[2], sem).wait()
        r = lax.rem(c, 16)
        cnt = jnp.zeros((16,), jnp.float32)
        for k in range(4):
            cnt = cnt + jnp.where(xt_blk[c, pl.ds(16 * k, 16)] != 1,
                                  ones16, zeros16)
        inv = ones16 / jnp.full((16,), jnp.sum(cnt))
        for j in range(19):
            off = 284 if j == 18 else 16 * j
            rows = rows3[off // 128]
            loff = off % 128

            def body(i, carry, rows=rows, loff=loff):
                acc = list(carry)
                for u in range(10):
                    v = rows[i * 10 + u, pl.ds(loff, 16)]
                    k = u % 4
                    acc[k] = acc[k] + v
                    acc[4 + k] = jnp.maximum(acc[4 + k], v)
                    acc[8 + k] = jnp.minimum(acc[8 + k], v)
                return tuple(acc)

            ninf16 = jnp.full((16,), -jnp.inf, jnp.float32)
            pinf16 = jnp.full((16,), jnp.inf, jnp.float32)
            init = (zeros16,) * 4 + (ninf16,) * 4 + (pinf16,) * 4
            acc = lax.fori_loop(0, _S // 10, body, init)
            a_s = (acc[0] + acc[1]) + (acc[2] + acc[3])
            a_mx = jnp.maximum(jnp.maximum(acc[4], acc[5]),
                               jnp.maximum(acc[6], acc[7]))
            a_mn = jnp.minimum(jnp.minimum(acc[8], acc[9]),
                               jnp.minimum(acc[10], acc[11]))
            out_blk[r, pl.ds(off, 16)] = a_mx
            out_blk[r, pl.ds(384 + off, 16)] = a_s * inv
            out_blk[r, pl.ds(768 + off, 16)] = a_mn

    rows_a3 = (rows_a0, rows_a1, rows_a2)
    rows_b3 = (rows_b0, rows_b1, rows_b2)
    fetch(0, rows_a3, sem_a)

    def loop_body(it, carry):
        c0 = 2 * it
        fetch(c0 + 1, rows_b3, sem_b)
        compute(c0, rows_a3, sem_a)

        @pl.when(it < _COLS // 2 - 1)
        def _():
            fetch(c0 + 2, rows_a3, sem_a)

        compute(c0 + 1, rows_b3, sem_b)

        @pl.when(lax.rem(it, 8) == 7)
        def _():
            grp = lax.div(it, 8)
            pltpu.sync_copy(out_blk, out_hbm.at[pl.ds(base + grp * 16, 16)])

        return carry

    lax.fori_loop(0, _COLS // 2, loop_body, 0)


def _sc_pool(t0, t1, t2, xtp, xgp):
    mesh = plsc.VectorSubcoreMesh(core_axis_name="c", subcore_axis_name="s")
    f = pl.kernel(
        _sc_pool_body,
        out_type=jax.ShapeDtypeStruct((_B, _PD), jnp.float32),
        mesh=mesh,
        compiler_params=pltpu.CompilerParams(use_tc_tiling_on_sc=False,
                                             needs_layout_passes=False),
        scratch_types=[
            pltpu.VMEM((_COLS, _SP), jnp.int32),
            pltpu.VMEM((_COLS, _S), jnp.int32),
            pltpu.VMEM((_S, 128), jnp.float32),
            pltpu.VMEM((_S, 128), jnp.float32),
            pltpu.VMEM((_S, 128), jnp.float32),
            pltpu.VMEM((_S, 128), jnp.float32),
            pltpu.VMEM((_S, 128), jnp.float32),
            pltpu.VMEM((_S, 128), jnp.float32),
            pltpu.VMEM((16, _PD), jnp.float32),
            pltpu.SemaphoreType.DMA,
            pltpu.SemaphoreType.DMA,
        ],
    )
    return f(t0, t1, t2, xtp, xgp)


def _tc_transpose_body(i_ref, o0_ref, o1_ref, o2_ref):
    o0_ref[...] = i_ref[pl.ds(0, 128)].T
    o1_ref[...] = i_ref[pl.ds(128, 128)].T
    o2_ref[...] = jnp.concatenate(
        [i_ref[pl.ds(256, 44)],
         jnp.zeros((84, 512), jnp.float32)], axis=0).T


def _tc_transpose(tt):
    spec = pl.BlockSpec((512, 128), lambda i: (i, 0))
    return pl.pallas_call(
        _tc_transpose_body,
        grid=(196,),
        in_specs=[pl.BlockSpec((_D, 512), lambda i: (0, i))],
        out_specs=[spec, spec, spec],
        out_shape=[jax.ShapeDtypeStruct((_VP, 128), jnp.float32)] * 3,
    )(tt)


def _tc_head_body(p_ref, ag_ref, w1_ref, w2_ref, b_ref, o_ref):
    acc = jnp.dot(p_ref[...], w1_ref[...], preferred_element_type=jnp.float32)
    acc = acc + jnp.dot(ag_ref[...], w2_ref[...],
                        preferred_element_type=jnp.float32)
    acc = acc + b_ref[...]
    cols = lax.broadcasted_iota(jnp.int32, acc.shape, 1)
    acc = jnp.where(cols < 10, acc, -jnp.inf)
    m = jnp.max(acc, axis=1, keepdims=True)
    lse = jnp.log(jnp.sum(jnp.exp(acc - m), axis=1, keepdims=True)) + m
    o_ref[...] = acc - lse


def _tc_head(pooled, ag, w1, w2, bp):
    return pl.pallas_call(
        _tc_head_body,
        grid=(16,),
        in_specs=[
            pl.BlockSpec((_B // 16, _PD), lambda i: (i, 0)),
            pl.BlockSpec((_B // 16, 128), lambda i: (i, 0)),
            pl.BlockSpec((_PD, 128), lambda i: (0, 0)),
            pl.BlockSpec((128, 128), lambda i: (0, 0)),
            pl.BlockSpec((1, 128), lambda i: (0, 0)),
        ],
        out_specs=pl.BlockSpec((_B // 16, 128), lambda i: (i, 0)),
        out_shape=jax.ShapeDtypeStruct((_B, 128), jnp.float32),
    )(pooled, ag, w1, w2, bp)


def kernel(x, age, gender, table, W, b):
    xt = x.T
    xtp = jnp.full((_B, _SP), 1, jnp.int32).at[:, :_S].set(xt)
    t0, t1, t2 = _tc_transpose(jnp.swapaxes(table, 0, 1))
    pooled = _sc_pool(t0, t1, t2, xtp, xt)
    ag = (jnp.zeros((_B, 128), jnp.float32)
          .at[:, :11].set(age).at[:, 11:13].set(gender))
    w1 = (jnp.zeros((_PD, 128), jnp.float32)
          .at[0:300, :10].set(W[:, 0:300].T)
          .at[384:684, :10].set(W[:, 300:600].T)
          .at[768:1068, :10].set(W[:, 600:900].T))
    w2 = jnp.zeros((128, 128), jnp.float32).at[:13, :10].set(W[:, 900:].T)
    bp = jnp.zeros((1, 128), jnp.float32).at[0, :10].set(b)
    out = _tc_head(pooled, ag, w1, w2, bp)
    return out[:, :10]


# speedup vs baseline: 1.2997x; 1.1260x over previous
import jax
import jax.numpy as jnp
from jax import lax
from jax.experimental import pallas as pl
from jax.experimental.pallas import tpu as pltpu
from jax.experimental.pallas import tpu_sc as plsc

_S = 50
_B = 4096
_D = 300
_SP = 64
_NC = 2
_NS = 16
_NW = _NC * _NS
_COLS = _B // _NW
_PD = 1152
_VP = 98 * 1024


def _sc_pool_body(t0_hbm, t1_hbm, t2_hbm, xt_hbm, xg_hbm, out_hbm,
                  xt_blk, xg_blk, rows_a0, rows_a1, rows_a2,
                  rows_b0, rows_b1, rows_b2, out_blk,
                  sem_a, sem_b):
    wid = lax.axis_index("s") * _NC + lax.axis_index("c")
    base = wid * _COLS
    zeros16 = jnp.zeros((16,), jnp.float32)
    ones16 = jnp.full((16,), 1.0, jnp.float32)

    pltpu.sync_copy(xt_hbm.at[pl.ds(base, _COLS)], xt_blk)
    pltpu.sync_copy(xg_hbm.at[pl.ds(base, _COLS)], xg_blk)
    zoffs = [s + d for s in (0, 384, 768)
             for d in (300, 316, 332, 348, 364, 368)]
    for r in range(16):
        for z in zoffs:
            out_blk[r, pl.ds(z, 16)] = zeros16

    def fetch(c, rows3, sem):
        idx = xg_blk.at[c]
        pltpu.make_async_copy(t0_hbm.at[idx], rows3[0], sem).start()
        pltpu.make_async_copy(t1_hbm.at[idx], rows3[1], sem).start()
        pltpu.make_async_copy(t2_hbm.at[idx], rows3[2], sem).start()

    def compute(c, rows3, sem):
        idx = xg_blk.at[c]
        pltpu.make_async_copy(t0_hbm.at[idx], rows3[0], sem).wait()
        pltpu.make_async_copy(t1_hbm.at[idx], rows3[1], sem).wait()
        pltpu.make_async_copy(t2_hbm.at[idx], rows3[2], sem).wait()
        r = lax.rem(c, 16)
        cnt = jnp.zeros((16,), jnp.float32)
        for k in range(4):
            cnt = cnt + jnp.where(xt_blk[c, pl.ds(16 * k, 16)] != 1,
                                  ones16, zeros16)
        inv = ones16 / jnp.full((16,), jnp.sum(cnt))
        for j in range(19):
            off = 284 if j == 18 else 16 * j
            rows = rows3[off // 128]
            loff = off % 128

            def body(i, carry, rows=rows, loff=loff):
                acc = list(carry)
                for u in range(10):
                    v = rows[i * 10 + u, pl.ds(loff, 16)]
                    k = u % 4
                    acc[k] = acc[k] + v
                    acc[4 + k] = jnp.maximum(acc[4 + k], v)
                    acc[8 + k] = jnp.minimum(acc[8 + k], v)
                return tuple(acc)

            ninf16 = jnp.full((16,), -jnp.inf, jnp.float32)
            pinf16 = jnp.full((16,), jnp.inf, jnp.float32)
            init = (zeros16,) * 4 + (ninf16,) * 4 + (pinf16,) * 4
            acc = lax.fori_loop(0, _S // 10, body, init)
            a_s = (acc[0] + acc[1]) + (acc[2] + acc[3])
            a_mx = jnp.maximum(jnp.maximum(acc[4], acc[5]),
                               jnp.maximum(acc[6], acc[7]))
            a_mn = jnp.minimum(jnp.minimum(acc[8], acc[9]),
                               jnp.minimum(acc[10], acc[11]))
            out_blk[r, pl.ds(off, 16)] = a_mx
            out_blk[r, pl.ds(384 + off, 16)] = a_s * inv
            out_blk[r, pl.ds(768 + off, 16)] = a_mn

    rows_a3 = (rows_a0, rows_a1, rows_a2)
    rows_b3 = (rows_b0, rows_b1, rows_b2)
    fetch(0, rows_a3, sem_a)

    def loop_body(it, carry):
        c0 = 2 * it
        fetch(c0 + 1, rows_b3, sem_b)
        compute(c0, rows_a3, sem_a)

        @pl.when(it < _COLS // 2 - 1)
        def _():
            fetch(c0 + 2, rows_a3, sem_a)

        compute(c0 + 1, rows_b3, sem_b)

        @pl.when(lax.rem(it, 8) == 7)
        def _():
            grp = lax.div(it, 8)
            pltpu.sync_copy(out_blk, out_hbm.at[pl.ds(base + grp * 16, 16)])

        return carry

    lax.fori_loop(0, _COLS // 2, loop_body, 0)


def _sc_pool(t0, t1, t2, xtp, xgp):
    mesh = plsc.VectorSubcoreMesh(core_axis_name="c", subcore_axis_name="s")
    f = pl.kernel(
        _sc_pool_body,
        out_type=jax.ShapeDtypeStruct((_B, _PD), jnp.float32),
        mesh=mesh,
        compiler_params=pltpu.CompilerParams(use_tc_tiling_on_sc=False,
                                             needs_layout_passes=False),
        scratch_types=[
            pltpu.VMEM((_COLS, _SP), jnp.int32),
            pltpu.VMEM((_COLS, _S), jnp.int32),
            pltpu.VMEM((_S, 128), jnp.float32),
            pltpu.VMEM((_S, 128), jnp.float32),
            pltpu.VMEM((_S, 128), jnp.float32),
            pltpu.VMEM((_S, 128), jnp.float32),
            pltpu.VMEM((_S, 128), jnp.float32),
            pltpu.VMEM((_S, 128), jnp.float32),
            pltpu.VMEM((16, _PD), jnp.float32),
            pltpu.SemaphoreType.DMA,
            pltpu.SemaphoreType.DMA,
        ],
    )
    return f(t0, t1, t2, xtp, xgp)


def _tc_transpose_body(i_ref, o0_ref, o1_ref, o2_ref):
    o0_ref[...] = i_ref[pl.ds(0, 128)].T
    o1_ref[...] = i_ref[pl.ds(128, 128)].T
    o2_ref[...] = jnp.concatenate(
        [i_ref[pl.ds(256, 44)],
         jnp.zeros((84, 1024), jnp.float32)], axis=0).T


def _tc_transpose(tt):
    spec = pl.BlockSpec((1024, 128), lambda i: (i, 0))
    return pl.pallas_call(
        _tc_transpose_body,
        grid=(98,),
        in_specs=[pl.BlockSpec((_D, 1024), lambda i: (0, i))],
        out_specs=[spec, spec, spec],
        out_shape=[jax.ShapeDtypeStruct((_VP, 128), jnp.float32)] * 3,
    )(tt)


def _tc_head_body(p_ref, ag_ref, w1_ref, w2_ref, b_ref, o_ref):
    acc = jnp.dot(p_ref[...], w1_ref[...], preferred_element_type=jnp.float32)
    acc = acc + jnp.dot(ag_ref[...], w2_ref[...],
                        preferred_element_type=jnp.float32)
    acc = acc + b_ref[...]
    cols = lax.broadcasted_iota(jnp.int32, acc.shape, 1)
    acc = jnp.where(cols < 10, acc, -jnp.inf)
    m = jnp.max(acc, axis=1, keepdims=True)
    lse = jnp.log(jnp.sum(jnp.exp(acc - m), axis=1, keepdims=True)) + m
    o_ref[...] = acc - lse


def _tc_head(pooled, ag, w1, w2, bp):
    return pl.pallas_call(
        _tc_head_body,
        grid=(16,),
        in_specs=[
            pl.BlockSpec((_B // 16, _PD), lambda i: (i, 0)),
            pl.BlockSpec((_B // 16, 128), lambda i: (i, 0)),
            pl.BlockSpec((_PD, 128), lambda i: (0, 0)),
            pl.BlockSpec((128, 128), lambda i: (0, 0)),
            pl.BlockSpec((1, 128), lambda i: (0, 0)),
        ],
        out_specs=pl.BlockSpec((_B // 16, 128), lambda i: (i, 0)),
        out_shape=jax.ShapeDtypeStruct((_B, 128), jnp.float32),
    )(pooled, ag, w1, w2, bp)


def kernel(x, age, gender, table, W, b):
    xt = x.T
    xtp = jnp.full((_B, _SP), 1, jnp.int32).at[:, :_S].set(xt)
    t0, t1, t2 = _tc_transpose(jnp.swapaxes(table, 0, 1))
    pooled = _sc_pool(t0, t1, t2, xtp, xt)
    ag = (jnp.zeros((_B, 128), jnp.float32)
          .at[:, :11].set(age).at[:, 11:13].set(gender))
    w1 = (jnp.zeros((_PD, 128), jnp.float32)
          .at[0:300, :10].set(W[:, 0:300].T)
          .at[384:684, :10].set(W[:, 300:600].T)
          .at[768:1068, :10].set(W[:, 600:900].T))
    w2 = jnp.zeros((128, 128), jnp.float32).at[:13, :10].set(W[:, 900:].T)
    bp = jnp.zeros((1, 128), jnp.float32).at[0, :10].set(b)
    out = _tc_head(pooled, ag, w1, w2, bp)
    return out[:, :10]


# speedup vs baseline: 1.4035x; 1.0799x over previous
import jax
import jax.numpy as jnp
from jax import lax
from jax.experimental import pallas as pl
from jax.experimental.pallas import tpu as pltpu
from jax.experimental.pallas import tpu_sc as plsc

_S = 50
_B = 4096
_D = 300
_SP = 64
_NC = 2
_NS = 16
_NW = _NC * _NS
_COLS = _B // _NW
_PD = 1152
_VP = 49 * 2048


def _sc_pool_body(t0_hbm, t1_hbm, t2_hbm, xt_hbm, xg_hbm, out_hbm,
                  xt_blk, xg_blk, rows_a0, rows_a1, rows_a2,
                  rows_b0, rows_b1, rows_b2, out_blk,
                  sem_a, sem_b):
    wid = lax.axis_index("s") * _NC + lax.axis_index("c")
    base = wid * _COLS
    zeros16 = jnp.zeros((16,), jnp.float32)
    ones16 = jnp.full((16,), 1.0, jnp.float32)

    pltpu.sync_copy(xt_hbm.at[pl.ds(base, _COLS)], xt_blk)
    pltpu.sync_copy(xg_hbm.at[pl.ds(base, _COLS)], xg_blk)
    zoffs = [s + d for s in (0, 384, 768)
             for d in (300, 316, 332, 348, 364, 368)]
    for r in range(16):
        for z in zoffs:
            out_blk[r, pl.ds(z, 16)] = zeros16

    def fetch(c, rows3, sem):
        idx = xg_blk.at[c]
        pltpu.make_async_copy(t0_hbm.at[idx], rows3[0], sem).start()
        pltpu.make_async_copy(t1_hbm.at[idx], rows3[1], sem).start()
        pltpu.make_async_copy(t2_hbm.at[idx], rows3[2], sem).start()

    def compute(c, rows3, sem):
        idx = xg_blk.at[c]
        pltpu.make_async_copy(t0_hbm.at[idx], rows3[0], sem).wait()
        pltpu.make_async_copy(t1_hbm.at[idx], rows3[1], sem).wait()
        pltpu.make_async_copy(t2_hbm.at[idx], rows3[2], sem).wait()
        r = lax.rem(c, 16)
        cnt = jnp.zeros((16,), jnp.float32)
        for k in range(4):
            cnt = cnt + jnp.where(xt_blk[c, pl.ds(16 * k, 16)] != 1,
                                  ones16, zeros16)
        inv = ones16 / jnp.full((16,), jnp.sum(cnt))
        for j in range(19):
            off = 284 if j == 18 else 16 * j
            rows = rows3[off // 128]
            loff = off % 128

            def body(i, carry, rows=rows, loff=loff):
                acc = list(carry)
                for u in range(10):
                    v = rows[i * 10 + u, pl.ds(loff, 16)]
                    k = u % 4
                    acc[k] = acc[k] + v
                    acc[4 + k] = jnp.maximum(acc[4 + k], v)
                    acc[8 + k] = jnp.minimum(acc[8 + k], v)
                return tuple(acc)

            ninf16 = jnp.full((16,), -jnp.inf, jnp.float32)
            pinf16 = jnp.full((16,), jnp.inf, jnp.float32)
            init = (zeros16,) * 4 + (ninf16,) * 4 + (pinf16,) * 4
            acc = lax.fori_loop(0, _S // 10, body, init)
            a_s = (acc[0] + acc[1]) + (acc[2] + acc[3])
            a_mx = jnp.maximum(jnp.maximum(acc[4], acc[5]),
                               jnp.maximum(acc[6], acc[7]))
            a_mn = jnp.minimum(jnp.minimum(acc[8], acc[9]),
                               jnp.minimum(acc[10], acc[11]))
            out_blk[r, pl.ds(off, 16)] = a_mx
            out_blk[r, pl.ds(384 + off, 16)] = a_s * inv
            out_blk[r, pl.ds(768 + off, 16)] = a_mn

    rows_a3 = (rows_a0, rows_a1, rows_a2)
    rows_b3 = (rows_b0, rows_b1, rows_b2)
    fetch(0, rows_a3, sem_a)

    def loop_body(it, carry):
        c0 = 2 * it
        fetch(c0 + 1, rows_b3, sem_b)
        compute(c0, rows_a3, sem_a)

        @pl.when(it < _COLS // 2 - 1)
        def _():
            fetch(c0 + 2, rows_a3, sem_a)

        compute(c0 + 1, rows_b3, sem_b)

        @pl.when(lax.rem(it, 8) == 7)
        def _():
            grp = lax.div(it, 8)
            pltpu.sync_copy(out_blk, out_hbm.at[pl.ds(base + grp * 16, 16)])

        return carry

    lax.fori_loop(0, _COLS // 2, loop_body, 0)


def _sc_pool(t0, t1, t2, xtp, xgp):
    mesh = plsc.VectorSubcoreMesh(core_axis_name="c", subcore_axis_name="s")
    f = pl.kernel(
        _sc_pool_body,
        out_type=jax.ShapeDtypeStruct((_B, _PD), jnp.float32),
        mesh=mesh,
        compiler_params=pltpu.CompilerParams(use_tc_tiling_on_sc=False,
                                             needs_layout_passes=False),
        scratch_types=[
            pltpu.VMEM((_COLS, _SP), jnp.int32),
            pltpu.VMEM((_COLS, _S), jnp.int32),
            pltpu.VMEM((_S, 128), jnp.float32),
            pltpu.VMEM((_S, 128), jnp.float32),
            pltpu.VMEM((_S, 128), jnp.float32),
            pltpu.VMEM((_S, 128), jnp.float32),
            pltpu.VMEM((_S, 128), jnp.float32),
            pltpu.VMEM((_S, 128), jnp.float32),
            pltpu.VMEM((16, _PD), jnp.float32),
            pltpu.SemaphoreType.DMA,
            pltpu.SemaphoreType.DMA,
        ],
    )
    return f(t0, t1, t2, xtp, xgp)


def _tc_transpose_body(i_ref, o0_ref, o1_ref, o2_ref):
    o0_ref[...] = i_ref[pl.ds(0, 128)].T
    o1_ref[...] = i_ref[pl.ds(128, 128)].T
    o2_ref[...] = jnp.concatenate(
        [i_ref[pl.ds(256, 44)],
         jnp.zeros((84, 2048), jnp.float32)], axis=0).T


def _tc_transpose(tt):
    spec = pl.BlockSpec((2048, 128), lambda i: (i, 0))
    return pl.pallas_call(
        _tc_transpose_body,
        grid=(49,),
        in_specs=[pl.BlockSpec((_D, 2048), lambda i: (0, i))],
        out_specs=[spec, spec, spec],
        out_shape=[jax.ShapeDtypeStruct((_VP, 128), jnp.float32)] * 3,
    )(tt)


def _tc_head_body(p_ref, ag_ref, w1_ref, w2_ref, b_ref, o_ref):
    acc = jnp.dot(p_ref[...], w1_ref[...], preferred_element_type=jnp.float32)
    acc = acc + jnp.dot(ag_ref[...], w2_ref[...],
                        preferred_element_type=jnp.float32)
    acc = acc + b_ref[...]
    cols = lax.broadcasted_iota(jnp.int32, acc.shape, 1)
    acc = jnp.where(cols < 10, acc, -jnp.inf)
    m = jnp.max(acc, axis=1, keepdims=True)
    lse = jnp.log(jnp.sum(jnp.exp(acc - m), axis=1, keepdims=True)) + m
    o_ref[...] = acc - lse


def _tc_head(pooled, ag, w1, w2, bp):
    return pl.pallas_call(
        _tc_head_body,
        grid=(16,),
        in_specs=[
            pl.BlockSpec((_B // 16, _PD), lambda i: (i, 0)),
            pl.BlockSpec((_B // 16, 128), lambda i: (i, 0)),
            pl.BlockSpec((_PD, 128), lambda i: (0, 0)),
            pl.BlockSpec((128, 128), lambda i: (0, 0)),
            pl.BlockSpec((1, 128), lambda i: (0, 0)),
        ],
        out_specs=pl.BlockSpec((_B // 16, 128), lambda i: (i, 0)),
        out_shape=jax.ShapeDtypeStruct((_B, 128), jnp.float32),
    )(pooled, ag, w1, w2, bp)


def kernel(x, age, gender, table, W, b):
    xt = x.T
    xtp = jnp.full((_B, _SP), 1, jnp.int32).at[:, :_S].set(xt)
    t0, t1, t2 = _tc_transpose(jnp.swapaxes(table, 0, 1))
    pooled = _sc_pool(t0, t1, t2, xtp, xt)
    ag = (jnp.zeros((_B, 128), jnp.float32)
          .at[:, :11].set(age).at[:, 11:13].set(gender))
    w1 = (jnp.zeros((_PD, 128), jnp.float32)
          .at[0:300, :10].set(W[:, 0:300].T)
          .at[384:684, :10].set(W[:, 300:600].T)
          .at[768:1068, :10].set(W[:, 600:900].T))
    w2 = jnp.zeros((128, 128), jnp.float32).at[:13, :10].set(W[:, 900:].T)
    bp = jnp.zeros((1, 128), jnp.float32).at[0, :10].set(b)
    out = _tc_head(pooled, ag, w1, w2, bp)
    return out[:, :10]


# speedup vs baseline: 1.4323x; 1.0205x over previous
import jax
import jax.numpy as jnp
from jax import lax
from jax.experimental import pallas as pl
from jax.experimental.pallas import tpu as pltpu
from jax.experimental.pallas import tpu_sc as plsc

_S = 50
_B = 4096
_D = 300
_SP = 64
_NC = 2
_NS = 16
_NW = _NC * _NS
_COLS = _B // _NW
_PD = 1152
_VP = 25 * 4096


def _sc_pool_body(t0_hbm, t1_hbm, t2_hbm, xt_hbm, xg_hbm, out_hbm,
                  xt_blk, xg_blk, rows_a0, rows_a1, rows_a2,
                  rows_b0, rows_b1, rows_b2, out_blk,
                  sem_a, sem_b):
    wid = lax.axis_index("s") * _NC + lax.axis_index("c")
    base = wid * _COLS
    zeros16 = jnp.zeros((16,), jnp.float32)
    ones16 = jnp.full((16,), 1.0, jnp.float32)

    pltpu.sync_copy(xt_hbm.at[pl.ds(base, _COLS)], xt_blk)
    pltpu.sync_copy(xg_hbm.at[pl.ds(base, _COLS)], xg_blk)
    zoffs = [s + d for s in (0, 384, 768)
             for d in (300, 316, 332, 348, 364, 368)]
    for r in range(16):
        for z in zoffs:
            out_blk[r, pl.ds(z, 16)] = zeros16

    def fetch(c, rows3, sem):
        idx = xg_blk.at[c]
        pltpu.make_async_copy(t0_hbm.at[idx], rows3[0], sem).start()
        pltpu.make_async_copy(t1_hbm.at[idx], rows3[1], sem).start()
        pltpu.make_async_copy(t2_hbm.at[idx], rows3[2], sem).start()

    def compute(c, rows3, sem):
        idx = xg_blk.at[c]
        pltpu.make_async_copy(t0_hbm.at[idx], rows3[0], sem).wait()
        pltpu.make_async_copy(t1_hbm.at[idx], rows3[1], sem).wait()
        pltpu.make_async_copy(t2_hbm.at[idx], rows3[2], sem).wait()
        r = lax.rem(c, 16)
        cnt = jnp.zeros((16,), jnp.float32)
        for k in range(4):
            cnt = cnt + jnp.where(xt_blk[c, pl.ds(16 * k, 16)] != 1,
                                  ones16, zeros16)
        inv = ones16 / jnp.full((16,), jnp.sum(cnt))
        for j in range(19):
            off = 284 if j == 18 else 16 * j
            rows = rows3[off // 128]
            loff = off % 128

            def body(i, carry, rows=rows, loff=loff):
                acc = list(carry)
                for u in range(10):
                    v = rows[i * 10 + u, pl.ds(loff, 16)]
                    k = u % 4
                    acc[k] = acc[k] + v
                    acc[4 + k] = jnp.maximum(acc[4 + k], v)
                    acc[8 + k] = jnp.minimum(acc[8 + k], v)
                return tuple(acc)

            ninf16 = jnp.full((16,), -jnp.inf, jnp.float32)
            pinf16 = jnp.full((16,), jnp.inf, jnp.float32)
            init = (zeros16,) * 4 + (ninf16,) * 4 + (pinf16,) * 4
            acc = lax.fori_loop(0, _S // 10, body, init)
            a_s = (acc[0] + acc[1]) + (acc[2] + acc[3])
            a_mx = jnp.maximum(jnp.maximum(acc[4], acc[5]),
                               jnp.maximum(acc[6], acc[7]))
            a_mn = jnp.minimum(jnp.minimum(acc[8], acc[9]),
                               jnp.minimum(acc[10], acc[11]))
            out_blk[r, pl.ds(off, 16)] = a_mx
            out_blk[r, pl.ds(384 + off, 16)] = a_s * inv
            out_blk[r, pl.ds(768 + off, 16)] = a_mn

    rows_a3 = (rows_a0, rows_a1, rows_a2)
    rows_b3 = (rows_b0, rows_b1, rows_b2)
    fetch(0, rows_a3, sem_a)

    def loop_body(it, carry):
        c0 = 2 * it
        fetch(c0 + 1, rows_b3, sem_b)
        compute(c0, rows_a3, sem_a)

        @pl.when(it < _COLS // 2 - 1)
        def _():
            fetch(c0 + 2, rows_a3, sem_a)

        compute(c0 + 1, rows_b3, sem_b)

        @pl.when(lax.rem(it, 8) == 7)
        def _():
            grp = lax.div(it, 8)
            pltpu.sync_copy(out_blk, out_hbm.at[pl.ds(base + grp * 16, 16)])

        return carry

    lax.fori_loop(0, _COLS // 2, loop_body, 0)


def _sc_pool(t0, t1, t2, xtp, xgp):
    mesh = plsc.VectorSubcoreMesh(core_axis_name="c", subcore_axis_name="s")
    f = pl.kernel(
        _sc_pool_body,
        out_type=jax.ShapeDtypeStruct((_B, _PD), jnp.float32),
        mesh=mesh,
        compiler_params=pltpu.CompilerParams(use_tc_tiling_on_sc=False,
                                             needs_layout_passes=False),
        scratch_types=[
            pltpu.VMEM((_COLS, _SP), jnp.int32),
            pltpu.VMEM((_COLS, _S), jnp.int32),
            pltpu.VMEM((_S, 128), jnp.float32),
            pltpu.VMEM((_S, 128), jnp.float32),
            pltpu.VMEM((_S, 128), jnp.float32),
            pltpu.VMEM((_S, 128), jnp.float32),
            pltpu.VMEM((_S, 128), jnp.float32),
            pltpu.VMEM((_S, 128), jnp.float32),
            pltpu.VMEM((16, _PD), jnp.float32),
            pltpu.SemaphoreType.DMA,
            pltpu.SemaphoreType.DMA,
        ],
    )
    return f(t0, t1, t2, xtp, xgp)


def _tc_transpose_body(i_ref, o0_ref, o1_ref, o2_ref):
    o0_ref[...] = i_ref[pl.ds(0, 128)].T
    o1_ref[...] = i_ref[pl.ds(128, 128)].T
    o2_ref[...] = jnp.concatenate(
        [i_ref[pl.ds(256, 44)],
         jnp.zeros((84, 4096), jnp.float32)], axis=0).T


def _tc_transpose(tt):
    spec = pl.BlockSpec((4096, 128), lambda i: (i, 0))
    return pl.pallas_call(
        _tc_transpose_body,
        grid=(25,),
        in_specs=[pl.BlockSpec((_D, 4096), lambda i: (0, i))],
        out_specs=[spec, spec, spec],
        out_shape=[jax.ShapeDtypeStruct((_VP, 128), jnp.float32)] * 3,
    )(tt)


def _tc_head_body(p_ref, ag_ref, w1_ref, w2_ref, b_ref, o_ref):
    acc = jnp.dot(p_ref[...], w1_ref[...], preferred_element_type=jnp.float32)
    acc = acc + jnp.dot(ag_ref[...], w2_ref[...],
                        preferred_element_type=jnp.float32)
    acc = acc + b_ref[...]
    cols = lax.broadcasted_iota(jnp.int32, acc.shape, 1)
    acc = jnp.where(cols < 10, acc, -jnp.inf)
    m = jnp.max(acc, axis=1, keepdims=True)
    lse = jnp.log(jnp.sum(jnp.exp(acc - m), axis=1, keepdims=True)) + m
    o_ref[...] = acc - lse


def _tc_head(pooled, ag, w1, w2, bp):
    return pl.pallas_call(
        _tc_head_body,
        grid=(16,),
        in_specs=[
            pl.BlockSpec((_B // 16, _PD), lambda i: (i, 0)),
            pl.BlockSpec((_B // 16, 128), lambda i: (i, 0)),
            pl.BlockSpec((_PD, 128), lambda i: (0, 0)),
            pl.BlockSpec((128, 128), lambda i: (0, 0)),
            pl.BlockSpec((1, 128), lambda i: (0, 0)),
        ],
        out_specs=pl.BlockSpec((_B // 16, 128), lambda i: (i, 0)),
        out_shape=jax.ShapeDtypeStruct((_B, 128), jnp.float32),
    )(pooled, ag, w1, w2, bp)


def kernel(x, age, gender, table, W, b):
    xt = x.T
    xtp = jnp.full((_B, _SP), 1, jnp.int32).at[:, :_S].set(xt)
    t0, t1, t2 = _tc_transpose(jnp.swapaxes(table, 0, 1))
    pooled = _sc_pool(t0, t1, t2, xtp, xt)
    ag = (jnp.zeros((_B, 128), jnp.float32)
          .at[:, :11].set(age).at[:, 11:13].set(gender))
    w1 = (jnp.zeros((_PD, 128), jnp.float32)
          .at[0:300, :10].set(W[:, 0:300].T)
          .at[384:684, :10].set(W[:, 300:600].T)
          .at[768:1068, :10].set(W[:, 600:900].T))
    w2 = jnp.zeros((128, 128), jnp.float32).at[:13, :10].set(W[:, 900:].T)
    bp = jnp.zeros((1, 128), jnp.float32).at[0, :10].set(b)
    out = _tc_head(pooled, ag, w1, w2, bp)
    return out[:, :10]
